# baseline (speedup 1.0000x reference)
import functools

import jax
import jax.numpy as jnp
from jax import lax
from jax.experimental import pallas as pl
from jax.experimental.pallas import tpu as pltpu
from jax.experimental.pallas import tpu_sc as plsc

B = 4096
S = 50
P = 20
T = P + S
D = 128

_info = plsc.get_sparse_core_info()
NC, NS = _info.num_cores, _info.num_subcores
NW = NC * NS
RW = B // NW
NBUF = 5
G = 16


def _make_kernel():
    mesh = plsc.VectorSubcoreMesh(core_axis_name="c", subcore_axis_name="s")

    @functools.partial(
        pl.kernel,
        mesh=mesh,
        compiler_params=pltpu.CompilerParams(use_tc_tiling_on_sc=True),
        out_type=jax.ShapeDtypeStruct((T, B, D), jnp.float32),
        scratch_types=[
            pltpu.VMEM((S, RW), jnp.int32),
            pltpu.VMEM((P, G, D), jnp.float32),
            *([pltpu.VMEM((RW, D), jnp.float32)] * 5),
            *([pltpu.SemaphoreType.DMA] * 11),
        ],
    )
    def k(ids_hbm, table_hbm, prompt_hbm, out_hbm,
          idx_v, pbuf, b0, b1, b2, b3, b4,
          g0, g1, g2, g3, g4, w0, w1, w2, w3, w4, psem):
        bufs = (b0, b1, b2, b3, b4)
        gsems = (g0, g1, g2, g3, g4)
        wsems = (w0, w1, w2, w3, w4)
        wid = lax.axis_index("s") * NC + lax.axis_index("c")
        base = wid * RW

        pltpu.sync_copy(ids_hbm.at[:, pl.ds(base, RW)], idx_v)

        for j in range(G):
            pltpu.async_copy(prompt_hbm, pbuf.at[:, pl.ds(j, 1)], psem)
        for j in range(G):
            pltpu.make_async_copy(
                prompt_hbm, pbuf.at[:, pl.ds(0, 1)], psem).wait()
        for g in range(RW // G):
            pltpu.async_copy(
                pbuf, out_hbm.at[pl.ds(0, P), pl.ds(base + g * G, G)], psem)

        def drain_gather(s):
            pltpu.make_async_copy(
                table_hbm.at[pl.ds(0, RW)], bufs[s], gsems[s]).wait()

        def fire_write(t, s):
            pltpu.async_copy(
                bufs[s], out_hbm.at[t, pl.ds(base, RW)], wsems[s])

        def drain_write(s):
            pltpu.make_async_copy(
                bufs[s], out_hbm.at[0, pl.ds(0, RW)], wsems[s]).wait()

        def fire_gather(t, s):
            pltpu.async_copy(
                table_hbm.at[idx_v.at[t - P]], bufs[s], gsems[s])

        for s in range(NBUF):
            fire_gather(P + s, s)

        def outer(c, carry):
            tt = P + c * NBUF
            for s in range(NBUF):
                drain_gather(s)
                fire_write(tt + s, s)
            for s in range(NBUF):
                drain_write(s)
                fire_gather(tt + NBUF + s, s)
            return carry

        lax.fori_loop(0, S // NBUF - 1, outer, 0)

        for s in range(NBUF):
            drain_gather(s)
            fire_write(T - NBUF + s, s)
        for s in range(NBUF):
            drain_write(s)

        for g in range(RW // G):
            pltpu.make_async_copy(
                pbuf, out_hbm.at[pl.ds(0, P), pl.ds(0, G)], psem).wait()

    return k


_kernel = _make_kernel()


def kernel(input_ids, embed_table, prompt_weight):
    ids_t = input_ids.astype(jnp.int32).T
    out = _kernel(ids_t, embed_table, prompt_weight.reshape(P, 1, D))
    return out.transpose(1, 0, 2)
